# baseline (speedup 1.0000x reference)
import functools

import jax
import jax.numpy as jnp
from jax import lax
from jax.experimental import pallas as pl
from jax.experimental.pallas import tpu as pltpu
from jax.experimental.pallas import tpu_sc as plsc


def _sc_gather(table, idx):
    info = plsc.get_sparse_core_info()
    num_workers = info.num_cores * info.num_subcores
    b = idx.shape[0]
    d = table.shape[1]
    b_per_w = b // num_workers
    mesh = plsc.VectorSubcoreMesh(core_axis_name="c", subcore_axis_name="s")

    @functools.partial(
        pl.kernel,
        mesh=mesh,
        compiler_params=pltpu.CompilerParams(use_tc_tiling_on_sc=False),
        out_type=jax.ShapeDtypeStruct((b, d), jnp.float32),
        scratch_types=[
            pltpu.VMEM((b_per_w,), jnp.int32),
            pltpu.VMEM((b_per_w, d), jnp.float32),
            pltpu.SemaphoreType.DMA,
        ],
    )
    def k(table_hbm, idx_hbm, out_hbm, idx_v, rows_v, sem):
        wid = lax.axis_index("s") * info.num_cores + lax.axis_index("c")
        base = wid * b_per_w
        pltpu.sync_copy(idx_hbm.at[pl.ds(base, b_per_w)], idx_v)
        pltpu.async_copy(table_hbm.at[idx_v], rows_v, sem).wait()
        pltpu.sync_copy(rows_v, out_hbm.at[pl.ds(base, b_per_w)])

    return k(table, idx)


def _sc_fused(x, idp, po_embed):
    info = plsc.get_sparse_core_info()
    num_workers = info.num_cores * info.num_subcores
    b, l, f = x.shape
    e = 64
    rows_per_w = b // num_workers
    ch = 40
    nch = l // ch
    n_iter = rows_per_w * nch
    mesh = plsc.VectorSubcoreMesh(core_axis_name="c", subcore_axis_name="s")

    @functools.partial(
        pl.kernel,
        mesh=mesh,
        out_type=jax.ShapeDtypeStruct((b, l, f + e), jnp.float32),
        scratch_types=[
            pltpu.VMEM((2, ch, f + e), jnp.float32),
            pltpu.VMEM((l, e), jnp.float32),
            pltpu.VMEM((rows_per_w, 2 * e), jnp.float32),
            pltpu.SemaphoreType.DMA,
            pltpu.SemaphoreType.DMA((2,)),
        ],
    )
    def k(x_hbm, idp_hbm, po_hbm, out_hbm, stage_v, po_v, id_v, sem_in, sem_out):
        wid = lax.axis_index("s") * info.num_cores + lax.axis_index("c")
        base_b = wid * rows_per_w
        pltpu.sync_copy(po_hbm, po_v)
        pltpu.sync_copy(idp_hbm.at[pl.ds(base_b, rows_per_w)], id_v)

        @pl.loop(0, n_iter)
        def _(i):
            buf = lax.rem(i, 2)
            rr = lax.div(i, nch)
            r = base_b + rr
            l0 = lax.rem(i, nch) * ch

            @pl.when(i >= 2)
            def _():
                pltpu.make_async_copy(
                    stage_v.at[buf],
                    out_hbm.at[0, pl.ds(0, ch)],
                    sem_out.at[buf],
                ).wait()

            in_cp = pltpu.async_copy(
                x_hbm.at[r, pl.ds(l0, ch)],
                stage_v.at[buf, :, pl.ds(0, f)],
                sem_in,
            )
            for c in range(e // 16):
                idvec = id_v[rr, pl.ds(16 * c, 16)]
                for j in range(ch):
                    stage_v[buf, j, pl.ds(f + 16 * c, 16)] = (
                        po_v[l0 + j, pl.ds(16 * c, 16)] + idvec
                    )
            in_cp.wait()
            pltpu.async_copy(
                stage_v.at[buf],
                out_hbm.at[r, pl.ds(l0, ch)],
                sem_out.at[buf],
            )

        for buf in (0, 1):
            pltpu.make_async_copy(
                stage_v.at[buf],
                out_hbm.at[0, pl.ds(0, ch)],
                sem_out.at[buf],
            ).wait()

    return k(x, idp, po_embed)


def kernel(series_id, x, id_embed, po_embed):
    id_rows = _sc_gather(id_embed, series_id.astype(jnp.int32))
    idp = jnp.concatenate([id_rows, id_rows], axis=1)
    return _sc_fused(x, idp, po_embed)


# speedup vs baseline: 1.2382x; 1.2382x over previous
import functools

import jax
import jax.numpy as jnp
from jax import lax
from jax.experimental import pallas as pl
from jax.experimental.pallas import tpu as pltpu
from jax.experimental.pallas import tpu_sc as plsc


def _sc_gather(table, idx):
    info = plsc.get_sparse_core_info()
    num_workers = info.num_cores * info.num_subcores
    b = idx.shape[0]
    d = table.shape[1]
    b_per_w = b // num_workers
    mesh = plsc.VectorSubcoreMesh(core_axis_name="c", subcore_axis_name="s")

    @functools.partial(
        pl.kernel,
        mesh=mesh,
        compiler_params=pltpu.CompilerParams(use_tc_tiling_on_sc=False),
        out_type=jax.ShapeDtypeStruct((b, d), jnp.float32),
        scratch_types=[
            pltpu.VMEM((b_per_w,), jnp.int32),
            pltpu.VMEM((b_per_w, d), jnp.float32),
            pltpu.SemaphoreType.DMA,
        ],
    )
    def k(table_hbm, idx_hbm, out_hbm, idx_v, rows_v, sem):
        wid = lax.axis_index("s") * info.num_cores + lax.axis_index("c")
        base = wid * b_per_w
        pltpu.sync_copy(idx_hbm.at[pl.ds(base, b_per_w)], idx_v)
        pltpu.async_copy(table_hbm.at[idx_v], rows_v, sem).wait()
        pltpu.sync_copy(rows_v, out_hbm.at[pl.ds(base, b_per_w)])

    return k(table, idx)


def _tc_fuse(x, id_rows, po_embed, batch_tile=32):
    b, l, f = x.shape
    e = po_embed.shape[1]

    def body(x_ref, id_ref, po_ref, out_ref):
        emb = po_ref[...][None, :, :] + id_ref[...][:, None, :]
        out_ref[...] = jnp.concatenate([x_ref[...], emb, emb], axis=2)

    return pl.pallas_call(
        body,
        grid=(b // batch_tile,),
        in_specs=[
            pl.BlockSpec((batch_tile, l, f), lambda i: (i, 0, 0)),
            pl.BlockSpec((batch_tile, e), lambda i: (i, 0)),
            pl.BlockSpec((l, e), lambda i: (0, 0)),
        ],
        out_specs=pl.BlockSpec((batch_tile, l, f + 2 * e), lambda i: (i, 0, 0)),
        out_shape=jax.ShapeDtypeStruct((b, l, f + e), jnp.float32),
    )(x, id_rows, po_embed)


def kernel(series_id, x, id_embed, po_embed):
    id_rows = _sc_gather(id_embed, series_id.astype(jnp.int32))
    return _tc_fuse(x, id_rows, po_embed)


# speedup vs baseline: 2.5159x; 2.0319x over previous
import functools

import jax
import jax.numpy as jnp
from jax import lax
from jax.experimental import pallas as pl
from jax.experimental.pallas import tpu as pltpu
from jax.experimental.pallas import tpu_sc as plsc


def _sc_gather(table, idx):
    info = plsc.get_sparse_core_info()
    num_workers = info.num_cores * info.num_subcores
    b = idx.shape[0]
    d = table.shape[1]
    b_per_w = b // num_workers
    mesh = plsc.VectorSubcoreMesh(core_axis_name="c", subcore_axis_name="s")

    @functools.partial(
        pl.kernel,
        mesh=mesh,
        compiler_params=pltpu.CompilerParams(use_tc_tiling_on_sc=False),
        out_type=jax.ShapeDtypeStruct((b, d), jnp.float32),
        scratch_types=[
            pltpu.VMEM((b_per_w,), jnp.int32),
            pltpu.VMEM((b_per_w, d), jnp.float32),
            pltpu.SemaphoreType.DMA,
        ],
    )
    def k(table_hbm, idx_hbm, out_hbm, idx_v, rows_v, sem):
        wid = lax.axis_index("s") * info.num_cores + lax.axis_index("c")
        base = wid * b_per_w
        pltpu.sync_copy(idx_hbm.at[pl.ds(base, b_per_w)], idx_v)
        pltpu.async_copy(table_hbm.at[idx_v], rows_v, sem).wait()
        pltpu.sync_copy(rows_v, out_hbm.at[pl.ds(base, b_per_w)])

    return k(table, idx)


def _tc_fuse_t(x, id_t, po3, l_tile=8):
    b, l, f = x.shape
    e = po3.shape[1]

    def body(x_ref, id_ref, po_ref, out_ref):
        for j in range(l_tile):
            out_ref[j, 0:f, :] = x_ref[:, j, :].T
            out_ref[j, f:, :] = po_ref[j, :, :] + id_ref[...]

    return pl.pallas_call(
        body,
        grid=(l // l_tile,),
        in_specs=[
            pl.BlockSpec((b, l_tile, f), lambda i: (0, i, 0)),
            pl.BlockSpec((e, b), lambda i: (0, 0)),
            pl.BlockSpec((l_tile, e, 1), lambda i: (i, 0, 0)),
        ],
        out_specs=pl.BlockSpec((l_tile, f + e, b), lambda i: (i, 0, 0)),
        out_shape=jax.ShapeDtypeStruct((l, f + e, b), jnp.float32),
    )(x, id_t, po3)


def kernel(series_id, x, id_embed, po_embed):
    id_rows = _sc_gather(id_embed, series_id.astype(jnp.int32))
    out_t = _tc_fuse_t(x, id_rows.T, po_embed[:, :, None])
    return jnp.transpose(out_t, (2, 0, 1))


# speedup vs baseline: 2.5175x; 1.0006x over previous
import functools

import jax
import jax.numpy as jnp
from jax import lax
from jax.experimental import pallas as pl
from jax.experimental.pallas import tpu as pltpu
from jax.experimental.pallas import tpu_sc as plsc


def _sc_gather(table, idx):
    info = plsc.get_sparse_core_info()
    num_workers = info.num_cores * info.num_subcores
    b = idx.shape[0]
    d = table.shape[1]
    b_per_w = b // num_workers
    mesh = plsc.VectorSubcoreMesh(core_axis_name="c", subcore_axis_name="s")

    @functools.partial(
        pl.kernel,
        mesh=mesh,
        compiler_params=pltpu.CompilerParams(use_tc_tiling_on_sc=False),
        out_type=jax.ShapeDtypeStruct((b, d), jnp.float32),
        scratch_types=[
            pltpu.VMEM((b_per_w,), jnp.int32),
            pltpu.VMEM((b_per_w, d), jnp.float32),
            pltpu.SemaphoreType.DMA,
        ],
    )
    def k(table_hbm, idx_hbm, out_hbm, idx_v, rows_v, sem):
        wid = lax.axis_index("s") * info.num_cores + lax.axis_index("c")
        base = wid * b_per_w
        pltpu.sync_copy(idx_hbm.at[pl.ds(base, b_per_w)], idx_v)
        pltpu.async_copy(table_hbm.at[idx_v], rows_v, sem).wait()
        pltpu.sync_copy(rows_v, out_hbm.at[pl.ds(base, b_per_w)])

    return k(table, idx)


def _tc_xpart(x, l_tile=16):
    b, l, f = x.shape

    def body(x_ref, out_ref):
        for j in range(l_tile):
            out_ref[j, :, :] = x_ref[:, j, :].T

    return pl.pallas_call(
        body,
        grid=(l // l_tile,),
        in_specs=[pl.BlockSpec((b, l_tile, f), lambda i: (0, i, 0))],
        out_specs=pl.BlockSpec((l_tile, f, b), lambda i: (i, 0, 0)),
        out_shape=jax.ShapeDtypeStruct((l, f + 64, b), jnp.float32),
    )(x)


def _tc_embpart(y, id_t, po3, l_tile=16):
    l, w, b = y.shape
    e = w - 128

    def body(y_ref, id_ref, po_ref, out_ref):
        del y_ref
        for j in range(l_tile):
            out_ref[j, :, :] = po_ref[j, :, :] + id_ref[...]

    return pl.pallas_call(
        body,
        grid=(l // l_tile,),
        in_specs=[
            pl.BlockSpec(memory_space=pl.ANY),
            pl.BlockSpec((e, b), lambda i: (0, 0)),
            pl.BlockSpec((l_tile, e, 1), lambda i: (i, 0, 0)),
        ],
        out_specs=pl.BlockSpec((l_tile, e, b), lambda i: (i, 2, 0)),
        out_shape=jax.ShapeDtypeStruct((l, w, b), jnp.float32),
        input_output_aliases={0: 0},
    )(y, id_t, po3)


def kernel(series_id, x, id_embed, po_embed):
    id_rows = _sc_gather(id_embed, series_id.astype(jnp.int32))
    y = _tc_xpart(x)
    out_t = _tc_embpart(y, id_rows.T, po_embed[:, :, None])
    return jnp.transpose(out_t, (2, 0, 1))
